# baseline (speedup 1.0000x reference)
import functools

import jax
import jax.numpy as jnp
from jax import lax
from jax.experimental import pallas as pl
from jax.experimental.pallas import tpu as pltpu
from jax.experimental.pallas import tpu_sc as plsc

_K = 8
_B, _T, _C = 4, 4096, 2048
_NW = 32
_CW = 128
_G = _CW // 16
_R = 512
_NCH = _T // _R
_UNITS = _B * (_C // _CW)
_UPW = _UNITS // _NW

_SORT8 = [
    (0, 1), (2, 3), (4, 5), (6, 7),
    (0, 2), (1, 3), (4, 6), (5, 7),
    (1, 2), (5, 6),
    (0, 4), (1, 5), (2, 6), (3, 7),
    (2, 4), (3, 5),
    (1, 2), (3, 4), (5, 6),
]
_BITONIC8 = [
    (0, 4), (1, 5), (2, 6), (3, 7),
    (0, 2), (1, 3), (4, 6), (5, 7),
    (0, 1), (2, 3), (4, 5), (6, 7),
]


def _cx(a, i, j):
    hi = jnp.maximum(a[i], a[j])
    lo = jnp.minimum(a[i], a[j])
    a[i] = hi
    a[j] = lo


def _merge8(acc, s):
    m = [jnp.maximum(acc[i], s[_K - 1 - i]) for i in range(_K)]
    for (i, j) in _BITONIC8:
        _cx(m, i, j)
    return m


def _sc_body(x_hbm, out_hbm, buf, obuf):
    wid = lax.axis_index("s") * 2 + lax.axis_index("c")
    neg = jnp.full((16,), -jnp.inf, dtype=jnp.float32)
    ngroups = _C // _CW
    for u in range(_UPW):
        unit = wid * _UPW + u
        b = unit // ngroups
        c0 = pl.multiple_of((unit % ngroups) * _CW, _CW)
        acc = [[neg for _ in range(_K)] for _ in range(_G)]
        for ch in range(_NCH):
            pltpu.sync_copy(
                x_hbm.at[b, pl.ds(ch * _R, _R), pl.ds(c0, _CW)], buf)
            for g in range(_G):
                def rowbody(r8, carry, _g=g):
                    s = [buf[r8 * _K + j, pl.ds(_g * 16, 16)]
                         for j in range(_K)]
                    for (i, j) in _SORT8:
                        _cx(s, i, j)
                    return tuple(_merge8(list(carry), s))

                acc[g] = list(
                    lax.fori_loop(0, _R // _K, rowbody, tuple(acc[g])))
        for g in range(_G):
            for r in range(_K):
                obuf[r, pl.ds(g * 16, 16)] = acc[g][r]
        pltpu.sync_copy(obuf, out_hbm.at[b, :, pl.ds(c0, _CW)])


def kernel(inputs):
    mesh = plsc.VectorSubcoreMesh(core_axis_name="c", subcore_axis_name="s")
    run = functools.partial(
        pl.kernel,
        out_type=jax.ShapeDtypeStruct((_B, _K, _C), jnp.float32),
        scratch_types=[
            pltpu.VMEM((_R, _CW), jnp.float32),
            pltpu.VMEM((_K, _CW), jnp.float32),
        ],
        mesh=mesh,
    )(_sc_body)
    out = run(inputs)
    return out.reshape(_B, _K * _C)


# speedup vs baseline: 2.5038x; 2.5038x over previous
import jax
import jax.numpy as jnp
from jax.experimental import pallas as pl
from jax.experimental.pallas import tpu as pltpu

_K = 8

_SORT8 = [
    (0, 1), (2, 3), (4, 5), (6, 7),
    (0, 2), (1, 3), (4, 6), (5, 7),
    (1, 2), (5, 6),
    (0, 4), (1, 5), (2, 6), (3, 7),
    (2, 4), (3, 5),
    (1, 2), (3, 4), (5, 6),
]

_BITONIC8 = [
    (0, 4), (1, 5), (2, 6), (3, 7),
    (0, 2), (1, 3), (4, 6), (5, 7),
    (0, 1), (2, 3), (4, 5), (6, 7),
]


def _cx(a, i, j):
    hi = jnp.maximum(a[i], a[j])
    lo = jnp.minimum(a[i], a[j])
    a[i] = hi
    a[j] = lo


def _merge8(acc, s):
    m = [jnp.maximum(acc[i], s[_K - 1 - i]) for i in range(_K)]
    for (i, j) in _BITONIC8:
        _cx(m, i, j)
    return m


def _topk_kernel(x_ref, o_ref, acc_ref):
    tb = x_ref.shape[1]
    chunks = tb // 64

    def chunk_sorted(base):
        s = [x_ref[0, pl.ds(base + _K * j, _K), :] for j in range(_K)]
        for (i, j) in _SORT8:
            _cx(s, i, j)
        return s

    stack = []
    for m in range(chunks):
        node = (0, chunk_sorted(m * 64))
        while stack and stack[-1][0] == node[0]:
            lvl, other = stack.pop()
            node = (lvl + 1, _merge8(other, node[1]))
        stack.append(node)
    a = stack[0][1]
    for _, other in stack[1:]:
        a = _merge8(other, a)

    t_id = pl.program_id(2)
    nt = pl.num_programs(2)

    @pl.when(t_id == 0)
    def _():
        for i in range(_K):
            acc_ref[i] = a[i]

    @pl.when(t_id != 0)
    def _():
        merged = _merge8([acc_ref[i] for i in range(_K)], a)
        for i in range(_K):
            acc_ref[i] = merged[i]

    @pl.when(t_id == nt - 1)
    def _():
        b = [acc_ref[i] for i in range(_K)]
        h = _K // 2
        while h >= 1:
            top = [v[:h, :] for v in b]
            bot = [v[h:2 * h, :] for v in b]
            b = _merge8(top, bot)
            h //= 2
        for i in range(_K):
            o_ref[0, i, :] = b[i][0]


def kernel(inputs):
    b, t, c = inputs.shape
    cb = 512
    tb = 1024
    out = pl.pallas_call(
        _topk_kernel,
        grid=(b, c // cb, t // tb),
        in_specs=[pl.BlockSpec((1, tb, cb), lambda i, j, k: (i, k, j))],
        out_specs=pl.BlockSpec((1, _K, cb), lambda i, j, k: (i, 0, j)),
        out_shape=jax.ShapeDtypeStruct((b, _K, c), inputs.dtype),
        scratch_shapes=[pltpu.VMEM((_K, _K, cb), inputs.dtype)],
    )(inputs)
    return out.reshape(b, _K * c)


# speedup vs baseline: 3.6738x; 1.4673x over previous
import jax
import jax.numpy as jnp
from jax.experimental import pallas as pl

_K = 8

_SORT8 = [
    (0, 1), (2, 3), (4, 5), (6, 7),
    (0, 2), (1, 3), (4, 6), (5, 7),
    (1, 2), (5, 6),
    (0, 4), (1, 5), (2, 6), (3, 7),
    (2, 4), (3, 5),
    (1, 2), (3, 4), (5, 6),
]

_BITONIC8 = [
    (0, 4), (1, 5), (2, 6), (3, 7),
    (0, 2), (1, 3), (4, 6), (5, 7),
    (0, 1), (2, 3), (4, 5), (6, 7),
]


def _cx(a, i, j):
    hi = jnp.maximum(a[i], a[j])
    lo = jnp.minimum(a[i], a[j])
    a[i] = hi
    a[j] = lo


def _merge8(acc, s):
    m = [jnp.maximum(acc[i], s[_K - 1 - i]) for i in range(_K)]
    for (i, j) in _BITONIC8:
        _cx(m, i, j)
    return m


def _topk_kernel(x_ref, o_ref):
    t = x_ref.shape[1]
    chunks = t // 64

    def chunk_sorted(base):
        s = [x_ref[0, pl.ds(base + _K * j, _K), :] for j in range(_K)]
        for (i, j) in _SORT8:
            _cx(s, i, j)
        return s

    stack = []
    for m in range(chunks):
        node = (0, chunk_sorted(m * 64))
        while stack and stack[-1][0] == node[0]:
            lvl, other = stack.pop()
            node = (lvl + 1, _merge8(other, node[1]))
        stack.append(node)
    a = stack[0][1]
    for _, other in stack[1:]:
        a = _merge8(other, a)
    h = _K // 2
    while h >= 1:
        top = [v[:h, :] for v in a]
        bot = [v[h:2 * h, :] for v in a]
        a = _merge8(top, bot)
        h //= 2
    for i in range(_K):
        o_ref[0, i, :] = a[i][0]


def kernel(inputs):
    b, t, c = inputs.shape
    cb = 512
    out = pl.pallas_call(
        _topk_kernel,
        grid=(b, c // cb),
        in_specs=[pl.BlockSpec((1, t, cb), lambda i, j: (i, 0, j))],
        out_specs=pl.BlockSpec((1, _K, cb), lambda i, j: (i, 0, j)),
        out_shape=jax.ShapeDtypeStruct((b, _K, c), inputs.dtype),
    )(inputs)
    return out.reshape(b, _K * c)
